# baseline (speedup 1.0000x reference)
import functools

import jax
import jax.numpy as jnp
from jax import lax
from jax.experimental import pallas as pl
from jax.experimental.pallas import tpu as pltpu
from jax.experimental.pallas import tpu_sc as plsc

N = 10000
E = 160000
IN_FEATS = 512
HID = 512
NCL = 128
K = 10
ALPHA = 0.1

NPAD = 10240
CH = 64
NBUF = 3
NCHUNK = 81
EPAD = 32 * NCHUNK * CH

_info = plsc.get_sparse_core_info()
NC = _info.num_cores
NS = _info.num_subcores
NW = NC * NS
RPT = NPAD // NS

RB = 1024
RB2 = 1280


def _mlp_body(x_ref, w0_ref, b0_ref, w1_ref, b1_ref, w2_ref, b2_ref, n_ref,
              u0_ref, c1_ref, c2_ref):
    h = jnp.dot(x_ref[...], w0_ref[...], preferred_element_type=jnp.float32)
    h = jnp.maximum(h + b0_ref[...], 0.0)
    h = jnp.dot(h, w1_ref[...], preferred_element_type=jnp.float32)
    h = jnp.maximum(h + b1_ref[...], 0.0)
    h = jnp.dot(h, w2_ref[...], preferred_element_type=jnp.float32)
    h = h + b2_ref[...]
    nn = n_ref[...]
    u0_ref[...] = nn * h
    c1_ref[...] = (ALPHA * nn) * h
    c2_ref[...] = ALPHA * h


_mlp = pl.pallas_call(
    _mlp_body,
    grid=(NPAD // RB,),
    in_specs=[
        pl.BlockSpec((RB, IN_FEATS), lambda i: (i, 0)),
        pl.BlockSpec((IN_FEATS, HID), lambda i: (0, 0)),
        pl.BlockSpec((1, HID), lambda i: (0, 0)),
        pl.BlockSpec((HID, HID), lambda i: (0, 0)),
        pl.BlockSpec((1, HID), lambda i: (0, 0)),
        pl.BlockSpec((HID, NCL), lambda i: (0, 0)),
        pl.BlockSpec((1, NCL), lambda i: (0, 0)),
        pl.BlockSpec((RB, 1), lambda i: (i, 0)),
    ],
    out_specs=[pl.BlockSpec((RB, NCL), lambda i: (i, 0))] * 3,
    out_shape=[jax.ShapeDtypeStruct((NPAD, NCL), jnp.float32)] * 3,
)


def _comb_body(s_ref, a_ref, c_ref, o_ref):
    o_ref[...] = a_ref[...] * (s_ref[0] + s_ref[1]) + c_ref[...]


_combine = pl.pallas_call(
    _comb_body,
    grid=(NPAD // RB2,),
    in_specs=[
        pl.BlockSpec((2, RB2, NCL), lambda i: (0, i, 0)),
        pl.BlockSpec((RB2, 1), lambda i: (i, 0)),
        pl.BlockSpec((RB2, NCL), lambda i: (i, 0)),
    ],
    out_specs=pl.BlockSpec((RB2, NCL), lambda i: (i, 0)),
    out_shape=jax.ShapeDtypeStruct((NPAD, NCL), jnp.float32),
)


@functools.partial(
    pl.kernel,
    out_type=jax.ShapeDtypeStruct((NC, NPAD, NCL), jnp.float32),
    mesh=plsc.VectorSubcoreMesh(core_axis_name="c", subcore_axis_name="s"),
    scratch_types=[
        pltpu.VMEM((NCHUNK, CH), jnp.int32),
        pltpu.VMEM((NCHUNK, CH), jnp.int32),
    ] + [pltpu.VMEM((CH, NCL), jnp.float32)] * NBUF
      + [pltpu.VMEM_SHARED((NPAD, NCL), jnp.float32)]
      + [pltpu.SemaphoreType.DMA] * NBUF,
)
def _sc_scatter(u_hbm, src_hbm, dst_hbm, zeros_hbm, out_hbm,
                src_v, dst_v, *rest):
    bufs = rest[:NBUF]
    acc = rest[NBUF]
    sems = rest[NBUF + 1:]
    c = lax.axis_index("c")
    s = lax.axis_index("s")
    wid = s * NC + c

    pltpu.sync_copy(src_hbm.at[wid], src_v)
    pltpu.sync_copy(dst_hbm.at[wid], dst_v)

    pltpu.sync_copy(zeros_hbm, acc.at[pl.ds(s * RPT, RPT)])
    plsc.subcore_barrier()

    for b in range(NBUF):
        pltpu.async_copy(u_hbm.at[src_v.at[b]], bufs[b], sems[b])

    def body(i, carry):
        j = i * NBUF
        for b in range(NBUF):
            pltpu.make_async_copy(u_hbm.at[src_v.at[0]], bufs[b], sems[b]).wait()
            pltpu.sync_copy(bufs[b], acc.at[dst_v.at[j + b]], add=True)
            nxt = jnp.minimum(j + b + NBUF, NCHUNK - 1)
            pltpu.async_copy(u_hbm.at[src_v.at[nxt]], bufs[b], sems[b])
        return carry

    lax.fori_loop(0, NCHUNK // NBUF, body, 0)
    for b in range(NBUF):
        pltpu.make_async_copy(u_hbm.at[src_v.at[0]], bufs[b], sems[b]).wait()
    plsc.subcore_barrier()

    pltpu.sync_copy(acc.at[pl.ds(s * RPT, RPT)],
                    out_hbm.at[c, pl.ds(s * RPT, RPT)])


def kernel(features, edge_index, W0, b0, W1, b1, W2, b2, norm):
    feats_p = jnp.pad(features, ((0, NPAD - N), (0, 0)))
    norm_p = jnp.pad(norm, ((0, NPAD - N), (0, 0)))
    ei_p = jnp.pad(edge_index, ((0, 0), (0, EPAD - E)), constant_values=N)
    src_pk = ei_p[0].reshape(NW, NCHUNK, CH)
    dst_pk = ei_p[1].reshape(NW, NCHUNK, CH)
    zeros = jnp.zeros((RPT, NCL), jnp.float32)
    a1 = (1.0 - ALPHA) * norm_p * norm_p
    a2 = (1.0 - ALPHA) * norm_p

    u, c1, c2 = _mlp(feats_p, W0, b0.reshape(1, HID), W1, b1.reshape(1, HID),
                     W2, b2.reshape(1, NCL), norm_p)
    for _ in range(K - 1):
        s_part = _sc_scatter(u, src_pk, dst_pk, zeros)
        u = _combine(s_part, a1, c1)
    s_part = _sc_scatter(u, src_pk, dst_pk, zeros)
    h = _combine(s_part, a2, c2)
    return h[:N]


# speedup vs baseline: 2.5974x; 2.5974x over previous
import functools

import jax
import jax.numpy as jnp
from jax import lax
from jax.experimental import pallas as pl
from jax.experimental.pallas import tpu as pltpu
from jax.experimental.pallas import tpu_sc as plsc

N = 10000
E = 160000
IN_FEATS = 512
HID = 512
NCL = 128
K = 10
ALPHA = 0.1

NPAD = 10240
NU = 10112
CH = 128
NPHASE = 8
QROWS = NPAD // NPHASE
ACCR = QROWS + 8
COH = 16
SLAB = COH + 8
PCH = 1288
PTOT = PCH * CH
DUMMY_SRC = N
DUMMY_DST = QROWS

_info = plsc.get_sparse_core_info()
NC = _info.num_cores
NS = _info.num_subcores
NW = NC * NS
URT = NU // NS
ZR = QROWS // NS
DR = QROWS // NS

RB = 1024
RB2 = 2048


def _mlp_body(x_ref, w0_ref, b0_ref, w1_ref, b1_ref, w2_ref, b2_ref, n_ref,
              u0_ref, c1_ref, c2_ref):
    h = jnp.dot(x_ref[...], w0_ref[...], preferred_element_type=jnp.float32)
    h = jnp.maximum(h + b0_ref[...], 0.0)
    h = jnp.dot(h.astype(jnp.bfloat16), w1_ref[...],
                preferred_element_type=jnp.float32)
    h = jnp.maximum(h + b1_ref[...], 0.0)
    h = jnp.dot(h.astype(jnp.bfloat16), w2_ref[...],
                preferred_element_type=jnp.float32)
    h = h + b2_ref[...]
    nn = n_ref[...]
    u0_ref[...] = nn * h
    c1_ref[...] = (ALPHA * nn) * h
    c2_ref[...] = ALPHA * h


_mlp = pl.pallas_call(
    _mlp_body,
    grid=(NPAD // RB,),
    in_specs=[
        pl.BlockSpec((RB, IN_FEATS), lambda i: (i, 0)),
        pl.BlockSpec((IN_FEATS, HID), lambda i: (0, 0)),
        pl.BlockSpec((1, HID), lambda i: (0, 0)),
        pl.BlockSpec((HID, HID), lambda i: (0, 0)),
        pl.BlockSpec((1, HID), lambda i: (0, 0)),
        pl.BlockSpec((HID, NCL), lambda i: (0, 0)),
        pl.BlockSpec((1, NCL), lambda i: (0, 0)),
        pl.BlockSpec((RB, 1), lambda i: (i, 0)),
    ],
    out_specs=[pl.BlockSpec((RB, NCL), lambda i: (i, 0))] * 3,
    out_shape=[jax.ShapeDtypeStruct((NPAD, NCL), jnp.float32)] * 3,
)


def _comb_body(s_ref, a_ref, c_ref, o_ref):
    o_ref[...] = a_ref[...] * s_ref[...] + c_ref[...]


_combine = pl.pallas_call(
    _comb_body,
    grid=(NPAD // RB2,),
    in_specs=[
        pl.BlockSpec((RB2, NCL), lambda i: (i, 0)),
        pl.BlockSpec((RB2, 1), lambda i: (i, 0)),
        pl.BlockSpec((RB2, NCL), lambda i: (i, 0)),
    ],
    out_specs=pl.BlockSpec((RB2, NCL), lambda i: (i, 0)),
    out_shape=jax.ShapeDtypeStruct((NPAD, NCL), jnp.float32),
)


@functools.partial(
    pl.kernel,
    out_type=jax.ShapeDtypeStruct((NPAD, NCL), jnp.float32),
    mesh=plsc.VectorSubcoreMesh(core_axis_name="c", subcore_axis_name="s"),
    scratch_types=[
        pltpu.VMEM((SLAB, CH), jnp.int32),
        pltpu.VMEM((SLAB, CH), jnp.int32),
        pltpu.VMEM((CH, NCL), jnp.float32),
        pltpu.VMEM((CH, NCL), jnp.float32),
        pltpu.VMEM_SHARED((NU, NCL), jnp.float32),
        pltpu.VMEM_SHARED((ACCR, NCL), jnp.float32),
        pltpu.VMEM((2 * NPHASE,), jnp.int32),
        pltpu.SemaphoreType.DMA,
        pltpu.SemaphoreType.DMA,
    ],
)
def _sc_scatter(u_hbm, ed_hbm, meta_hbm, zeros_hbm, out_hbm,
                didx, sidx, buf_a, buf_b, u_sp, acc, meta_vm, sem_a, sem_b):
    c = lax.axis_index("c")
    s = lax.axis_index("s")
    w = s * NC + c
    bufs = (buf_a, buf_b)
    sems = (sem_a, sem_b)

    pltpu.sync_copy(meta_hbm, meta_vm)
    u_off = pl.multiple_of(s * URT, 8)
    pltpu.sync_copy(u_hbm.at[pl.ds(u_off, URT)], u_sp.at[pl.ds(u_off, URT)])
    plsc.subcore_barrier()

    meta_v = meta_vm[...]

    def phase_body(q):
        nch = meta_v[q]
        pch = meta_v[NPHASE + q]
        active = c == (q % 2)
        lo = (nch * s) >> 4
        hi = (nch * (s + 1)) >> 4
        n_my = hi - lo
        ncoh = (n_my + (COH - 1)) >> 4

        @pl.when(active)
        def _():
            pltpu.sync_copy(zeros_hbm, acc.at[pl.ds(pl.multiple_of(s * ZR, 8), ZR)])
            plsc.subcore_barrier()

        def cohort_body(cb, carry1):
            g0 = pch + lo + cb * COH
            g8 = pl.multiple_of((g0 >> 3) << 3, 8)
            r0 = g0 - g8
            nj = jnp.minimum(COH, n_my - cb * COH)
            pltpu.sync_copy(ed_hbm.at[pl.ds(g8, SLAB)], didx)
            for ur in range(SLAB):
                for ug in range(CH // 16):
                    v = didx[ur, pl.ds(ug * 16, 16)]
                    sidx[ur, pl.ds(ug * 16, 16)] = jax.lax.shift_right_logical(v, 11)
                    didx[ur, pl.ds(ug * 16, 16)] = jax.lax.bitwise_and(v, 2047)
            pltpu.async_copy(u_sp.at[sidx.at[r0]], bufs[0], sems[0])
            for j in range(COH):
                if j + 1 < COH:
                    @pl.when(j + 1 < nj)
                    def _():
                        pltpu.async_copy(u_sp.at[sidx.at[r0 + j + 1]],
                                         bufs[(j + 1) % 2], sems[(j + 1) % 2])

                @pl.when(j < nj)
                def _():
                    pltpu.make_async_copy(u_hbm.at[pl.ds(0, CH)],
                                          bufs[j % 2], sems[j % 2]).wait()
                    pltpu.sync_copy(bufs[j % 2], acc.at[didx.at[r0 + j]], add=True)
            return carry1

        @pl.when(active)
        def _():
            lax.fori_loop(0, ncoh, cohort_body, 0)
            plsc.subcore_barrier()
            pltpu.sync_copy(acc.at[pl.ds(pl.multiple_of(s * DR, 8), DR)],
                            out_hbm.at[pl.ds(pl.multiple_of(q * QROWS + s * DR, 8), DR)])

    for q in range(NPHASE):
        phase_body(q)


def kernel(features, edge_index, W0, b0, W1, b1, W2, b2, norm):
    feats_p = jnp.pad(features, ((0, NPAD - N), (0, 0)))
    norm_p = jnp.pad(norm, ((0, NPAD - N), (0, 0)))
    a1 = (1.0 - ALPHA) * norm_p * norm_p
    a2 = (1.0 - ALPHA) * norm_p
    zeros = jnp.zeros((ZR, NCL), jnp.float32)

    src = edge_index[0]
    dst = edge_index[1]
    b = dst // QROWS
    order = jnp.argsort(b, stable=True)
    packed = src * 2048 + (dst - b * QROWS)
    packed_s = packed[order]
    cnt = jnp.bincount(b, length=NPHASE).astype(jnp.int32)
    start = jnp.concatenate([jnp.zeros((1,), jnp.int32),
                             jnp.cumsum(cnt)[:-1].astype(jnp.int32)])
    pc = ((cnt + (CH - 1)) // CH) * CH
    pstart = jnp.concatenate([jnp.zeros((1,), jnp.int32),
                              jnp.cumsum(pc)[:-1].astype(jnp.int32)])
    p = jnp.arange(PTOT, dtype=jnp.int32)
    qb = jnp.searchsorted(pstart, p, side="right").astype(jnp.int32) - 1
    qb = jnp.minimum(qb, NPHASE - 1)
    rel = p - pstart[qb]
    valid = rel < cnt[qb]
    g = jnp.where(valid, start[qb] + jnp.minimum(rel, cnt[qb] - 1), 0)
    pk_pad = jnp.where(valid, packed_s[g], DUMMY_SRC * 2048 + DUMMY_DST)
    ed_pk = pk_pad.reshape(PCH, CH)
    meta = jnp.concatenate([(pc // CH).astype(jnp.int32),
                            (pstart // CH).astype(jnp.int32)])

    u, c1, c2 = _mlp(feats_p.astype(jnp.bfloat16), W0.astype(jnp.bfloat16),
                     b0.reshape(1, HID), W1.astype(jnp.bfloat16),
                     b1.reshape(1, HID), W2.astype(jnp.bfloat16),
                     b2.reshape(1, NCL), norm_p)
    for _ in range(K - 1):
        s_part = _sc_scatter(u, ed_pk, meta, zeros)
        u = _combine(s_part, a1, c1)
    s_part = _sc_scatter(u, ed_pk, meta, zeros)
    h = _combine(s_part, a2, c2)
    return h[:N]
